# Initial kernel scaffold; baseline (speedup 1.0000x reference)
import jax
import jax.numpy as jnp
from jax.experimental import pallas as pl


def kernel(x, table):
    raise NotImplementedError("write your pallas kernel here")


# speedup vs baseline: 2.2825x; 2.2825x over previous
import jax
import jax.numpy as jnp
from jax.experimental import pallas as pl


def _add_kernel(x_ref, t_ref, o_ref):
    o_ref[...] = x_ref[...] + t_ref[...][None, :, :]


def kernel(x, table):
    B, S, H = x.shape
    TS = 256
    grid = (S // TS,)
    return pl.pallas_call(
        _add_kernel,
        grid=grid,
        in_specs=[
            pl.BlockSpec((B, TS, H), lambda j: (0, j, 0)),
            pl.BlockSpec((TS, H), lambda j: (j, 0)),
        ],
        out_specs=pl.BlockSpec((B, TS, H), lambda j: (0, j, 0)),
        out_shape=jax.ShapeDtypeStruct((B, S, H), x.dtype),
    )(x, table[:S])


# speedup vs baseline: 2.3563x; 1.0323x over previous
import jax
import jax.numpy as jnp
from jax.experimental import pallas as pl


def _add_kernel(x_ref, t_ref, o_ref):
    o_ref[...] = x_ref[...] + t_ref[...][None, :, :]


def kernel(x, table):
    B, S, H = x.shape
    TS = 512
    grid = (S // TS,)
    return pl.pallas_call(
        _add_kernel,
        grid=grid,
        in_specs=[
            pl.BlockSpec((B, TS, H), lambda j: (0, j, 0)),
            pl.BlockSpec((TS, H), lambda j: (j, 0)),
        ],
        out_specs=pl.BlockSpec((B, TS, H), lambda j: (0, j, 0)),
        out_shape=jax.ShapeDtypeStruct((B, S, H), x.dtype),
    )(x, table[:S])


# speedup vs baseline: 2.4450x; 1.0376x over previous
import jax
import jax.numpy as jnp
from jax.experimental import pallas as pl


def _add_kernel(x_ref, t_ref, o_ref):
    o_ref[...] = x_ref[...] + t_ref[...][None, :, :]


def kernel(x, table):
    B, S, H = x.shape
    TS = 1024
    grid = (S // TS,)
    return pl.pallas_call(
        _add_kernel,
        grid=grid,
        in_specs=[
            pl.BlockSpec((B, TS, H), lambda j: (0, j, 0)),
            pl.BlockSpec((TS, H), lambda j: (j, 0)),
        ],
        out_specs=pl.BlockSpec((B, TS, H), lambda j: (0, j, 0)),
        out_shape=jax.ShapeDtypeStruct((B, S, H), x.dtype),
    )(x, table[:S])
